# baseline (speedup 1.0000x reference)
import numpy as np
import jax
import jax.numpy as jnp
from jax.experimental import pallas as pl
from jax.experimental.pallas import tpu as pltpu

N = 4096
KIN = 512
F = 64
NCH = 6
ALPHA = 0.1
_H = N // 2

_INTERPRET = False

_XB = 256
_NXX = N // _XB
_AB = 64
_NAA = N // _AB
_PB = 64
_NPP = N // _PB
_CB = 128
_NC = N // _CB
_SEG1 = _NXX
_SEG2 = _SEG1 + _NAA
_T1 = _SEG2 + 2 * _NPP


def _mega_body(x_ref, w_ref, a_ref, p1_ref, p2_ref, p3_ref,
               hA_ref, hA2_ref, hA3_ref, hs1_ref, hs2_ref, hs3_ref,
               abf_ref, sbf_ref, rbf_ref):
    t = pl.program_id(0)

    @pl.when(t < _SEG1)
    def _():
        blk = jnp.dot(x_ref[...], w_ref[...],
                      preferred_element_type=jnp.float32)
        sbf_ref[pl.ds(t * _XB, _XB), :] = blk.astype(jnp.bfloat16)

    @pl.when((t >= _SEG1) & (t < _SEG2))
    def _():
        b = t - _SEG1
        abf = a_ref[...].astype(jnp.bfloat16)
        abf_ref[pl.ds(b * _AB, _AB), :] = abf
        hA_ref[pl.ds(b * _AB, _AB), :] = jnp.dot(
            abf, sbf_ref[...], preferred_element_type=jnp.float32)

    @pl.when(t >= _SEG2)
    def _():
        u = t - _SEG2

        @pl.when(u % 2 == 0)
        def _():
            p = u // 2
            rows = pl.ds(p * _PB, _PB)
            sbf = sbf_ref[...]
            hs1_ref[rows, :] = jnp.abs(
                jnp.dot(p1_ref[...].astype(jnp.bfloat16), sbf,
                        preferred_element_type=jnp.float32))
            hs2_ref[rows, :] = jnp.abs(
                jnp.dot(p2_ref[...].astype(jnp.bfloat16), sbf,
                        preferred_element_type=jnp.float32))
            hs3_ref[rows, :] = jnp.abs(
                jnp.dot(p3_ref[...].astype(jnp.bfloat16), sbf,
                        preferred_element_type=jnp.float32))

        @pl.when(u % 2 != 0)
        def _():
            c = u // 2

            @pl.when(c == 0)
            def _():
                rbf_ref[...] = hA_ref[...].astype(jnp.bfloat16)

            @pl.when(c == _NC)
            def _():
                rbf_ref[...] = hA2_ref[...].astype(jnp.bfloat16)

            rbf = rbf_ref[...]

            @pl.when(c < _NC)
            def _():
                rows = pl.ds(c * _CB, _CB)
                hA2_ref[rows, :] = jnp.dot(abf_ref[rows, :], rbf,
                                           preferred_element_type=jnp.float32)

            @pl.when(c >= _NC)
            def _():
                rows = pl.ds((c - _NC) * _CB, _CB)
                hA3_ref[rows, :] = jnp.dot(abf_ref[rows, :], rbf,
                                           preferred_element_type=jnp.float32)


def _mega_call(x, W, A, P1, P2, P3):
    full = pl.BlockSpec((N, F), lambda t: (0, 0))
    pspec = pl.BlockSpec(
        (_PB, N), lambda t: (jnp.clip((t - _SEG2 + 1) // 2, 0, _NPP - 1), 0))
    return pl.pallas_call(
        _mega_body,
        grid=(_T1,),
        in_specs=[
            pl.BlockSpec((_XB, KIN), lambda t: (jnp.minimum(t, _NXX - 1), 0)),
            pl.BlockSpec((KIN, F), lambda t: (0, 0)),
            pl.BlockSpec((_AB, N),
                         lambda t: (jnp.clip(t - _SEG1, 0, _NAA - 1), 0)),
            pspec, pspec, pspec,
        ],
        out_specs=[full] * 6,
        out_shape=[jax.ShapeDtypeStruct((N, F), jnp.float32)] * 6,
        scratch_shapes=[
            pltpu.VMEM((N, N), jnp.bfloat16),
            pltpu.VMEM((N, F), jnp.bfloat16),
            pltpu.VMEM((N, F), jnp.bfloat16),
        ],
        interpret=_INTERPRET,
    )(x, W, A, P1, P2, P3)


_G_np = np.zeros((NCH, NCH * F), np.float32)
_S_np = np.zeros((NCH * F, F), np.float32)
for _c in range(NCH):
    for _f in range(F):
        _m = NCH * _f + _c
        _G_np[_m // F, F * _c + _f] = 1.0
        _S_np[F * _c + _f, _m % F] = 1.0


def _epi_body(hA_ref, hA2_ref, hA3_ref, hs1_ref, hs2_ref, hs3_ref,
              arow_ref, g_ref, sel_ref, att_ref, hp_ref):
    def _rb(x):
        return x.astype(jnp.bfloat16).astype(jnp.float32)

    a1 = _rb(arow_ref[0:1, :])
    a2 = _rb(arow_ref[1:2, :])
    chan_refs = (hA_ref, hA2_ref, hA3_ref, hs1_ref, hs2_ref, hs3_ref)
    chans = tuple(r[...] for r in chan_refs)
    ws = []
    for r in chan_refs:
        ev = _rb(r[pl.Slice(0, _H, 2), :])
        od = _rb(r[pl.Slice(1, _H, 2), :])
        w = jnp.sum(ev * a1 + od * a2, axis=1, keepdims=True)
        ws.append(jnp.where(w >= 0, w, ALPHA * w))
    e = jnp.concatenate(ws, axis=1)
    m = jnp.max(e, axis=1, keepdims=True)
    ex = jnp.exp(e - m)
    att_bot = ex / jnp.sum(ex, axis=1, keepdims=True)
    att_top = jnp.full((_H, NCH), 1.0 / NCH, jnp.float32)
    att = jnp.concatenate([att_top, att_bot], axis=0)
    att_ref[...] = att
    attg = jnp.dot(att, g_ref[...], preferred_element_type=jnp.float32)
    c_all = jnp.concatenate(chans, axis=1)
    hp_ref[...] = jnp.dot(c_all * attg, sel_ref[...],
                          preferred_element_type=jnp.float32) * (1.0 / NCH)


def _epi_call(hA, hA2, hA3, hs1, hs2, hs3, arow, G, S):
    full = pl.BlockSpec((N, F), lambda: (0, 0))
    return pl.pallas_call(
        _epi_body,
        in_specs=[full] * 6 + [
            pl.BlockSpec((2, F), lambda: (0, 0)),
            pl.BlockSpec((NCH, NCH * F), lambda: (0, 0)),
            pl.BlockSpec((NCH * F, F), lambda: (0, 0)),
        ],
        out_specs=[
            pl.BlockSpec((N, NCH), lambda: (0, 0)),
            pl.BlockSpec((N, F), lambda: (0, 0)),
        ],
        out_shape=[
            jax.ShapeDtypeStruct((N, NCH), jnp.float32),
            jax.ShapeDtypeStruct((N, F), jnp.float32),
        ],
        interpret=_INTERPRET,
    )(hA, hA2, hA3, hs1, hs2, hs3, arow, G, S)


def kernel(input, A_nor, P_sct1, P_sct2, P_sct3, W, a):
    hA, hA2, hA3, hs1, hs2, hs3 = _mega_call(
        input, W, A_nor, P_sct1, P_sct2, P_sct3)
    arow = a.reshape(2, F)
    att, hp = _epi_call(hA, hA2, hA3, hs1, hs2, hs3,
                        arow, jnp.asarray(_G_np), jnp.asarray(_S_np))
    return (hp, att.reshape(N, NCH, 1))


# speedup vs baseline: 1.7788x; 1.7788x over previous
import numpy as np
import jax
import jax.numpy as jnp
from jax.experimental import pallas as pl
from jax.experimental.pallas import tpu as pltpu

N = 4096
KIN = 512
F = 64
NCH = 6
ALPHA = 0.1

_INTERPRET = False

_XB = 512
_NX = N // _XB
_AB = 256
_NA = N // _AB
_CB = 512
_NC = N // _CB
_T1 = _NX + _NA + 2 * _NC


def _chain_body(x_ref, w_ref, a_ref, s_ref, hA_ref, hA2_ref, hA3_ref,
                abf_ref, sbf_ref):
    t = pl.program_id(0)

    @pl.when(t < _NX)
    def _():
        blk = jnp.dot(x_ref[...], w_ref[...],
                      preferred_element_type=jnp.float32)
        s_ref[pl.ds(t * _XB, _XB), :] = blk

    @pl.when(t == _NX)
    def _():
        sbf_ref[...] = s_ref[...].astype(jnp.bfloat16)

    @pl.when((t >= _NX) & (t < _NX + _NA))
    def _():
        b = t - _NX
        abf = a_ref[...].astype(jnp.bfloat16)
        abf_ref[pl.ds(b * _AB, _AB), :] = abf
        hA_ref[pl.ds(b * _AB, _AB), :] = jnp.dot(
            abf, sbf_ref[...], preferred_element_type=jnp.float32)

    @pl.when(t == _NX + _NA)
    def _():
        sbf_ref[...] = hA_ref[...].astype(jnp.bfloat16)

    @pl.when((t >= _NX + _NA) & (t < _NX + _NA + _NC))
    def _():
        b = t - (_NX + _NA)
        rows = pl.ds(b * _CB, _CB)
        hA2_ref[rows, :] = jnp.dot(abf_ref[rows, :], sbf_ref[...],
                                   preferred_element_type=jnp.float32)

    @pl.when(t == _NX + _NA + _NC)
    def _():
        sbf_ref[...] = hA2_ref[...].astype(jnp.bfloat16)

    @pl.when(t >= _NX + _NA + _NC)
    def _():
        b = t - (_NX + _NA + _NC)
        rows = pl.ds(b * _CB, _CB)
        hA3_ref[rows, :] = jnp.dot(abf_ref[rows, :], sbf_ref[...],
                                   preferred_element_type=jnp.float32)


def _chain_call(x, W, A):
    full = pl.BlockSpec((N, F), lambda t: (0, 0))
    return pl.pallas_call(
        _chain_body,
        grid=(_T1,),
        in_specs=[
            pl.BlockSpec((_XB, KIN), lambda t: (jnp.minimum(t, _NX - 1), 0)),
            pl.BlockSpec((KIN, F), lambda t: (0, 0)),
            pl.BlockSpec((_AB, N),
                         lambda t: (jnp.clip(t - _NX, 0, _NA - 1), 0)),
        ],
        out_specs=[full, full, full, full],
        out_shape=[jax.ShapeDtypeStruct((N, F), jnp.float32)] * 4,
        scratch_shapes=[
            pltpu.VMEM((N, N), jnp.bfloat16),
            pltpu.VMEM((N, F), jnp.bfloat16),
        ],
        interpret=_INTERPRET,
    )(x, W, A)


_PB = 256
_NP = N // _PB

_G_np = np.zeros((NCH, NCH * F), np.float32)
_S_np = np.zeros((NCH * F, F), np.float32)
for _c in range(NCH):
    for _f in range(F):
        _m = NCH * _f + _c
        _G_np[_m // F, F * _c + _f] = 1.0
        _S_np[F * _c + _f, _m % F] = 1.0

_H = N // 2


def _sct_epi_body(p1_ref, p2_ref, p3_ref, s_ref, hA_ref, hA2_ref, hA3_ref,
                  arow_ref, g_ref, sel_ref, att_ref, hp_ref,
                  sbf_ref, hs1_ref, hs2_ref, hs3_ref):
    t = pl.program_id(0)

    @pl.when(t == 0)
    def _():
        sbf_ref[...] = s_ref[...].astype(jnp.bfloat16)

    @pl.when(t < _NP)
    def _():
        sbf = sbf_ref[...]
        rows = pl.ds(t * _PB, _PB)
        hs1_ref[rows, :] = jnp.abs(
            jnp.dot(p1_ref[...].astype(jnp.bfloat16), sbf,
                    preferred_element_type=jnp.float32))
        hs2_ref[rows, :] = jnp.abs(
            jnp.dot(p2_ref[...].astype(jnp.bfloat16), sbf,
                    preferred_element_type=jnp.float32))
        hs3_ref[rows, :] = jnp.abs(
            jnp.dot(p3_ref[...].astype(jnp.bfloat16), sbf,
                    preferred_element_type=jnp.float32))

    @pl.when(t == _NP)
    def _():
        def _rb(x):
            return x.astype(jnp.bfloat16).astype(jnp.float32)

        a1 = _rb(arow_ref[0:1, :])
        a2 = _rb(arow_ref[1:2, :])
        chan_refs = (hA_ref, hA2_ref, hA3_ref, hs1_ref, hs2_ref, hs3_ref)
        chans = tuple(r[...] for r in chan_refs)
        ws = []
        for r in chan_refs:
            ev = _rb(r[pl.Slice(0, _H, 2), :])
            od = _rb(r[pl.Slice(1, _H, 2), :])
            w = jnp.sum(ev * a1 + od * a2, axis=1, keepdims=True)
            ws.append(jnp.where(w >= 0, w, ALPHA * w))
        e = jnp.concatenate(ws, axis=1)
        m = jnp.max(e, axis=1, keepdims=True)
        ex = jnp.exp(e - m)
        att_bot = ex / jnp.sum(ex, axis=1, keepdims=True)
        att_top = jnp.full((_H, NCH), 1.0 / NCH, jnp.float32)
        att = jnp.concatenate([att_top, att_bot], axis=0)
        att_ref[...] = att
        attg = jnp.dot(att, g_ref[...], preferred_element_type=jnp.float32)
        c_all = jnp.concatenate(chans, axis=1)
        hp_ref[...] = jnp.dot(c_all * attg, sel_ref[...],
                              preferred_element_type=jnp.float32) * (1.0 / NCH)


def _sct_epi_call(P1, P2, P3, s, hA, hA2, hA3, arow, G, S):
    pspec = pl.BlockSpec((_PB, N), lambda t: (jnp.minimum(t, _NP - 1), 0))
    full = pl.BlockSpec((N, F), lambda t: (0, 0))
    return pl.pallas_call(
        _sct_epi_body,
        grid=(_NP + 1,),
        in_specs=[pspec, pspec, pspec, full, full, full, full,
                  pl.BlockSpec((2, F), lambda t: (0, 0)),
                  pl.BlockSpec((NCH, NCH * F), lambda t: (0, 0)),
                  pl.BlockSpec((NCH * F, F), lambda t: (0, 0))],
        out_specs=[
            pl.BlockSpec((N, NCH), lambda t: (0, 0)),
            pl.BlockSpec((N, F), lambda t: (0, 0)),
        ],
        out_shape=[
            jax.ShapeDtypeStruct((N, NCH), jnp.float32),
            jax.ShapeDtypeStruct((N, F), jnp.float32),
        ],
        scratch_shapes=[pltpu.VMEM((N, F), jnp.bfloat16)] +
                       [pltpu.VMEM((N, F), jnp.float32)] * 3,
        interpret=_INTERPRET,
    )(P1, P2, P3, s, hA, hA2, hA3, arow, G, S)


def kernel(input, A_nor, P_sct1, P_sct2, P_sct3, W, a):
    s, hA, hA2, hA3 = _chain_call(input, W, A_nor)
    arow = a.reshape(2, F)
    att, hp = _sct_epi_call(P_sct1, P_sct2, P_sct3, s, hA, hA2, hA3,
                            arow, jnp.asarray(_G_np), jnp.asarray(_S_np))
    return (hp, att.reshape(N, NCH, 1))
